# Initial kernel scaffold; baseline (speedup 1.0000x reference)
import functools

import jax
import jax.numpy as jnp
from jax import lax
from jax.experimental import pallas as pl
from jax.experimental.pallas import tpu as pltpu
from jax.experimental.pallas import tpu_sc as plsc

NC = 2
NS = 16
NW = NC * NS
LANES = 16
K = 80


def _tc_linear(x, W_lin, W_gcn):
    n, d = x.shape
    br = 2000

    def body(x_ref, wl_ref, wg_ref, hl_ref, xw_ref):
        xb = x_ref[...]
        dn = (((1,), (1,)), ((), ()))
        hl_ref[...] = lax.dot_general(xb, wl_ref[...], dn,
                                      preferred_element_type=jnp.float32)
        xw_ref[...] = lax.dot_general(xb, wg_ref[...], dn,
                                      preferred_element_type=jnp.float32)

    return pl.pallas_call(
        body,
        grid=(n // br,),
        in_specs=[
            pl.BlockSpec((br, d), lambda i: (i, 0)),
            pl.BlockSpec((d, d), lambda i: (0, 0)),
            pl.BlockSpec((d, d), lambda i: (0, 0)),
        ],
        out_specs=[
            pl.BlockSpec((br, d), lambda i: (i, 0)),
            pl.BlockSpec((br, d), lambda i: (i, 0)),
        ],
        out_shape=[
            jax.ShapeDtypeStruct((n, d), jnp.float32),
            jax.ShapeDtypeStruct((n, d), jnp.float32),
        ],
    )(x, W_lin, W_gcn)


def _dis_from(deg_ref):
    dp = deg_ref[...]
    deg = dp[0] + dp[1]
    return jnp.where(deg > 0, lax.rsqrt(deg), 0.0)


def _tc_scale(deg3, xw):
    n, d = xw.shape
    br = 2000

    def body(deg_ref, xw_ref, y_ref):
        y_ref[...] = xw_ref[...] * _dis_from(deg_ref)

    return pl.pallas_call(
        body,
        grid=(n // br,),
        in_specs=[
            pl.BlockSpec((2, br, 1), lambda i: (0, i, 0)),
            pl.BlockSpec((br, d), lambda i: (i, 0)),
        ],
        out_specs=pl.BlockSpec((br, d), lambda i: (i, 0)),
        out_shape=jax.ShapeDtypeStruct((n, d), jnp.float32),
    )(deg3, xw)


def _tc_combine(hl, deg3, z):
    n, d = hl.shape
    br = 2000

    def body(hl_ref, deg_ref, z_ref, out_ref):
        zp = z_ref[...]
        out_ref[...] = hl_ref[...] + _dis_from(deg_ref) * (zp[0] + zp[1])

    return pl.pallas_call(
        body,
        grid=(n // br,),
        in_specs=[
            pl.BlockSpec((br, d), lambda i: (i, 0)),
            pl.BlockSpec((2, br, 1), lambda i: (0, i, 0)),
            pl.BlockSpec((2, br, d), lambda i: (0, i, 0)),
        ],
        out_specs=pl.BlockSpec((br, d), lambda i: (i, 0)),
        out_shape=jax.ShapeDtypeStruct((n, d), jnp.float32),
    )(hl, deg3, z)


def _sc_deg(col3, ew3, zeros1):
    n = zeros1.shape[0]
    nchunks = col3.shape[1]
    mesh = plsc.VectorSubcoreMesh(core_axis_name="c", subcore_axis_name="s")

    @functools.partial(
        pl.kernel,
        out_type=jax.ShapeDtypeStruct((NC, n), jnp.float32),
        mesh=mesh,
        scratch_types=[
            pltpu.VMEM((nchunks, K), jnp.int32),
            pltpu.VMEM((nchunks, K), jnp.float32),
            pltpu.VMEM_SHARED((n,), jnp.float32),
        ],
    )
    def k(col_hbm, ew_hbm, z_hbm, out_hbm, cidx, ewv, acc):
        c = lax.axis_index("c")
        s = lax.axis_index("s")
        wid = s * NC + c

        @pl.when(s == 0)
        def _():
            pltpu.sync_copy(z_hbm, acc)

        plsc.subcore_barrier()
        pltpu.sync_copy(col_hbm.at[wid], cidx)
        pltpu.sync_copy(ew_hbm.at[wid], ewv)

        def chunk(i, carry):
            pltpu.sync_copy(ewv.at[i], acc.at[cidx.at[i]], add=True)
            return carry

        lax.fori_loop(0, nchunks, chunk, 0)
        plsc.subcore_barrier()

        @pl.when(s == 0)
        def _():
            pltpu.sync_copy(acc, out_hbm.at[c])

    return k(col3, ew3, zeros1)


def _sc_msg(row3, col3, ew3, y, zeros2):
    n, d = y.shape
    nchunks = row3.shape[1]
    rows_per_tile = n // NS
    mesh = plsc.VectorSubcoreMesh(core_axis_name="c", subcore_axis_name="s")

    @functools.partial(
        pl.kernel,
        out_type=jax.ShapeDtypeStruct((NC, n, d), jnp.float32),
        mesh=mesh,
        scratch_types=[
            pltpu.VMEM((nchunks, K), jnp.int32),
            pltpu.VMEM((nchunks, K), jnp.int32),
            pltpu.VMEM((nchunks, K), jnp.float32),
            pltpu.VMEM((K, d), jnp.float32),
            pltpu.VMEM_SHARED((n, d), jnp.float32),
            pltpu.SemaphoreType.DMA,
        ],
    )
    def k(row_hbm, col_hbm, ew_hbm, y_hbm, z2_hbm, out_hbm,
          ridx, cidx, ewv, rows, acc, sem):
        c = lax.axis_index("c")
        s = lax.axis_index("s")
        wid = s * NC + c
        rbase = s * rows_per_tile

        pltpu.sync_copy(z2_hbm.at[pl.ds(rbase, rows_per_tile)],
                        acc.at[pl.ds(rbase, rows_per_tile)])
        pltpu.sync_copy(row_hbm.at[wid], ridx)
        pltpu.sync_copy(col_hbm.at[wid], cidx)
        pltpu.sync_copy(ew_hbm.at[wid], ewv)
        plsc.subcore_barrier()

        def chunk(i, carry):
            pltpu.async_copy(y_hbm.at[ridx.at[i]], rows, sem).wait()

            def srow(r, cc):
                sc = ewv[i, r]
                for j in range(d // LANES):
                    sl = (r, pl.ds(j * LANES, LANES))
                    rows[sl] = rows[sl] * sc
                return cc

            lax.fori_loop(0, K, srow, 0)
            pltpu.sync_copy(rows, acc.at[cidx.at[i]], add=True)
            return carry

        lax.fori_loop(0, nchunks, chunk, 0)
        plsc.subcore_barrier()
        pltpu.sync_copy(acc.at[pl.ds(rbase, rows_per_tile)],
                        out_hbm.at[c, pl.ds(rbase, rows_per_tile)])

    return k(row3, col3, ew3, y, zeros2)


def kernel(x, adj_t, edge_weight, W_lin, W_gcn):
    n, d = x.shape
    e = edge_weight.shape[0]
    nchunks = e // (NW * K)

    row3 = adj_t[0].astype(jnp.int32).reshape(NW, nchunks, K)
    col3 = adj_t[1].astype(jnp.int32).reshape(NW, nchunks, K)
    ew3 = edge_weight.astype(jnp.float32).reshape(NW, nchunks, K)
    zeros1 = jnp.zeros((n,), jnp.float32)
    zeros2 = jnp.zeros((n, d), jnp.float32)

    hl, xw = _tc_linear(x, W_lin, W_gcn)
    deg_p = _sc_deg(col3, ew3, zeros1)
    deg3 = deg_p.reshape(NC, n, 1)
    y = _tc_scale(deg3, xw)
    z = _sc_msg(row3, col3, ew3, y, zeros2)
    return _tc_combine(hl, deg3, z)


# speedup vs baseline: 9.6713x; 9.6713x over previous
import functools

import jax
import jax.numpy as jnp
from jax import lax
from jax.experimental import pallas as pl
from jax.experimental.pallas import tpu as pltpu
from jax.experimental.pallas import tpu_sc as plsc

NC = 2
NS = 16
NW = NC * NS
LANES = 16
K = 80


def _tc_linear(x, W_lin, W_gcn):
    n, d = x.shape
    br = 2000

    def body(x_ref, wl_ref, wg_ref, hl_ref, xw_ref):
        xb = x_ref[...]
        dn = (((1,), (1,)), ((), ()))
        hl_ref[...] = lax.dot_general(xb, wl_ref[...], dn,
                                      preferred_element_type=jnp.float32)
        xw_ref[...] = lax.dot_general(xb, wg_ref[...], dn,
                                      preferred_element_type=jnp.float32)

    return pl.pallas_call(
        body,
        grid=(n // br,),
        in_specs=[
            pl.BlockSpec((br, d), lambda i: (i, 0)),
            pl.BlockSpec((d, d), lambda i: (0, 0)),
            pl.BlockSpec((d, d), lambda i: (0, 0)),
        ],
        out_specs=[
            pl.BlockSpec((br, d), lambda i: (i, 0)),
            pl.BlockSpec((br, d), lambda i: (i, 0)),
        ],
        out_shape=[
            jax.ShapeDtypeStruct((n, d), jnp.float32),
            jax.ShapeDtypeStruct((n, d), jnp.float32),
        ],
    )(x, W_lin, W_gcn)


def _dis_from(deg_ref):
    dp = deg_ref[...]
    deg = dp[0] + dp[1]
    return jnp.where(deg > 0, lax.rsqrt(deg), 0.0)


def _tc_scale(deg3, xw):
    n, d = xw.shape
    dh = d // NC
    br = 2000

    def body(deg_ref, xw_ref, y_ref):
        dis = _dis_from(deg_ref)
        xb = xw_ref[...]
        y_ref[0] = xb[:, :dh] * dis
        y_ref[1] = xb[:, dh:] * dis

    return pl.pallas_call(
        body,
        grid=(n // br,),
        in_specs=[
            pl.BlockSpec((2, br, 1), lambda i: (0, i, 0)),
            pl.BlockSpec((br, d), lambda i: (i, 0)),
        ],
        out_specs=pl.BlockSpec((NC, br, dh), lambda i: (0, i, 0)),
        out_shape=jax.ShapeDtypeStruct((NC, n, dh), jnp.float32),
    )(deg3, xw)


def _tc_combine(hl, deg3, z):
    n, d = hl.shape
    br = 2000

    def body(hl_ref, deg_ref, z_ref, out_ref):
        zp = z_ref[...]
        zc = jnp.concatenate([zp[0], zp[1]], axis=-1)
        out_ref[...] = hl_ref[...] + _dis_from(deg_ref) * zc

    return pl.pallas_call(
        body,
        grid=(n // br,),
        in_specs=[
            pl.BlockSpec((br, d), lambda i: (i, 0)),
            pl.BlockSpec((2, br, 1), lambda i: (0, i, 0)),
            pl.BlockSpec((2, br, d // NC), lambda i: (0, i, 0)),
        ],
        out_specs=pl.BlockSpec((br, d), lambda i: (i, 0)),
        out_shape=jax.ShapeDtypeStruct((n, d), jnp.float32),
    )(hl, deg3, z)


def _sc_deg(col3, ew3, zeros1):
    n = zeros1.shape[0]
    nchunks = col3.shape[1]
    mesh = plsc.VectorSubcoreMesh(core_axis_name="c", subcore_axis_name="s")

    @functools.partial(
        pl.kernel,
        out_type=jax.ShapeDtypeStruct((NC, n), jnp.float32),
        mesh=mesh,
        scratch_types=[
            pltpu.VMEM((nchunks, K), jnp.int32),
            pltpu.VMEM((nchunks, K), jnp.float32),
            pltpu.VMEM_SHARED((n,), jnp.float32),
        ],
        compiler_params=pltpu.CompilerParams(use_tc_tiling_on_sc=False),
    )
    def k(col_hbm, ew_hbm, z_hbm, out_hbm, cidx, ewv, acc):
        c = lax.axis_index("c")
        s = lax.axis_index("s")
        wid = s * NC + c

        @pl.when(s == 0)
        def _():
            pltpu.sync_copy(z_hbm, acc)

        plsc.subcore_barrier()
        pltpu.sync_copy(col_hbm.at[wid], cidx)
        pltpu.sync_copy(ew_hbm.at[wid], ewv)

        def chunk(i, carry):
            pltpu.sync_copy(ewv.at[i], acc.at[cidx.at[i]], add=True)
            return carry

        lax.fori_loop(0, nchunks, chunk, 0)
        plsc.subcore_barrier()

        @pl.when(s == 0)
        def _():
            pltpu.sync_copy(acc, out_hbm.at[c])

    return k(col3, ew3, zeros1)


def _sc_msg(row3, col3, ew3, y2, zeros2):
    _, n, dh = y2.shape
    np_ = zeros2.shape[0]
    nchunks = row3.shape[1]
    rows_per_tile = np_ // NS
    mesh = plsc.VectorSubcoreMesh(core_axis_name="c", subcore_axis_name="s")

    @functools.partial(
        pl.kernel,
        out_type=jax.ShapeDtypeStruct((NC, np_, dh), jnp.float32),
        mesh=mesh,
        scratch_types=[
            pltpu.VMEM((nchunks, K), jnp.int32),
            pltpu.VMEM((nchunks, K), jnp.int32),
            pltpu.VMEM((nchunks, K), jnp.float32),
            pltpu.VMEM((K, dh), jnp.float32),
            pltpu.VMEM_SHARED((np_, dh), jnp.float32),
            pltpu.SemaphoreType.DMA,
        ],
        compiler_params=pltpu.CompilerParams(use_tc_tiling_on_sc=False),
    )
    def k(row_hbm, col_hbm, ew_hbm, y_hbm, z2_hbm, out_hbm,
          ridx, cidx, ewv, rows, acc, sem):
        c = lax.axis_index("c")
        s = lax.axis_index("s")
        rbase = s * rows_per_tile

        pltpu.sync_copy(z2_hbm.at[pl.ds(rbase, rows_per_tile)],
                        acc.at[pl.ds(rbase, rows_per_tile)])
        pltpu.sync_copy(row_hbm.at[s], ridx)
        pltpu.sync_copy(col_hbm.at[s], cidx)
        pltpu.sync_copy(ew_hbm.at[s], ewv)
        plsc.subcore_barrier()

        def chunk(i, carry):
            pltpu.async_copy(y_hbm.at[c].at[ridx.at[i]], rows, sem).wait()

            def sgroup(g, cc):
                base = g * LANES
                sv = ewv[i, pl.ds(base, LANES)]
                for l in range(LANES):
                    sc = sv[l]
                    r = base + l
                    for j in range(dh // LANES):
                        sl = (r, pl.ds(j * LANES, LANES))
                        rows[sl] = rows[sl] * sc
                return cc

            lax.fori_loop(0, K // LANES, sgroup, 0)
            pltpu.sync_copy(rows, acc.at[cidx.at[i]], add=True)
            return carry

        lax.fori_loop(0, nchunks, chunk, 0)
        plsc.subcore_barrier()
        pltpu.sync_copy(acc.at[pl.ds(rbase, rows_per_tile)],
                        out_hbm.at[c, pl.ds(rbase, rows_per_tile)])

    return k(row3, col3, ew3, y2, zeros2)


def kernel(x, adj_t, edge_weight, W_lin, W_gcn):
    n, d = x.shape
    e = edge_weight.shape[0]
    nchunks = e // (NS * K)

    nchunks_deg = e // (NW * K)
    row_i = adj_t[0].astype(jnp.int32)
    col_i = adj_t[1].astype(jnp.int32)
    ew_f = edge_weight.astype(jnp.float32)
    col3d = col_i.reshape(NW, nchunks_deg, K)
    ew3d = ew_f.reshape(NW, nchunks_deg, K)
    row3 = row_i.reshape(NS, nchunks, K)
    col3 = col_i.reshape(NS, nchunks, K)
    ew3 = ew_f.reshape(NS, nchunks, K)
    zeros1 = jnp.zeros((n,), jnp.float32)
    npad = ((n + 8 * NS - 1) // (8 * NS)) * (8 * NS)
    zeros2 = jnp.zeros((npad, d // NC), jnp.float32)

    hl, xw = _tc_linear(x, W_lin, W_gcn)
    deg_p = _sc_deg(col3d, ew3d, zeros1)
    deg3 = deg_p.reshape(NC, n, 1)
    y2 = _tc_scale(deg3, xw)
    z = _sc_msg(row3, col3, ew3, y2, zeros2)
    return _tc_combine(hl, deg3, z)


# speedup vs baseline: 26.7414x; 2.7650x over previous
import functools

import jax
import jax.numpy as jnp
from jax import lax
from jax.experimental import pallas as pl
from jax.experimental.pallas import tpu as pltpu
from jax.experimental.pallas import tpu_sc as plsc

NC = 2
NS = 16
NW = NC * NS
LANES = 16
K = 80


def _tc_linear(x, W_lin, W_gcn):
    n, d = x.shape
    br = 2000

    def body(x_ref, wl_ref, wg_ref, hl_ref, xw_ref):
        xb = x_ref[...]
        dn = (((1,), (1,)), ((), ()))
        hl_ref[...] = lax.dot_general(xb, wl_ref[...], dn,
                                      preferred_element_type=jnp.float32)
        xw_ref[...] = lax.dot_general(xb, wg_ref[...], dn,
                                      preferred_element_type=jnp.float32)

    return pl.pallas_call(
        body,
        grid=(n // br,),
        in_specs=[
            pl.BlockSpec((br, d), lambda i: (i, 0)),
            pl.BlockSpec((d, d), lambda i: (0, 0)),
            pl.BlockSpec((d, d), lambda i: (0, 0)),
        ],
        out_specs=[
            pl.BlockSpec((br, d), lambda i: (i, 0)),
            pl.BlockSpec((br, d), lambda i: (i, 0)),
        ],
        out_shape=[
            jax.ShapeDtypeStruct((n, d), jnp.float32),
            jax.ShapeDtypeStruct((n, d), jnp.float32),
        ],
    )(x, W_lin, W_gcn)


def _dis_from(deg_ref):
    dp = deg_ref[...]
    deg = dp[0] + dp[1]
    return jnp.where(deg > 0, lax.rsqrt(deg), 0.0)


def _tc_scale(deg3, xw):
    n, d = xw.shape
    dh = d // NC
    br = 2000

    def body(deg_ref, xw_ref, y_ref):
        dis = _dis_from(deg_ref)
        xb = xw_ref[...]
        y_ref[0] = xb[:, :dh] * dis
        y_ref[1] = xb[:, dh:] * dis

    return pl.pallas_call(
        body,
        grid=(n // br,),
        in_specs=[
            pl.BlockSpec((2, br, 1), lambda i: (0, i, 0)),
            pl.BlockSpec((br, d), lambda i: (i, 0)),
        ],
        out_specs=pl.BlockSpec((NC, br, dh), lambda i: (0, i, 0)),
        out_shape=jax.ShapeDtypeStruct((NC, n, dh), jnp.float32),
    )(deg3, xw)


def _tc_combine(hl, deg3, z):
    n, d = hl.shape
    br = 2000

    def body(hl_ref, deg_ref, z_ref, out_ref):
        zp = z_ref[...]
        zc = jnp.concatenate([zp[0], zp[1]], axis=-1)
        out_ref[...] = hl_ref[...] + _dis_from(deg_ref) * zc

    return pl.pallas_call(
        body,
        grid=(n // br,),
        in_specs=[
            pl.BlockSpec((br, d), lambda i: (i, 0)),
            pl.BlockSpec((2, br, 1), lambda i: (0, i, 0)),
            pl.BlockSpec((2, br, d // NC), lambda i: (0, i, 0)),
        ],
        out_specs=pl.BlockSpec((br, d), lambda i: (i, 0)),
        out_shape=jax.ShapeDtypeStruct((n, d), jnp.float32),
    )(hl, deg3, z)


def _sc_deg(col3, ew3, zeros1):
    n = zeros1.shape[0]
    nchunks = col3.shape[1]
    mesh = plsc.VectorSubcoreMesh(core_axis_name="c", subcore_axis_name="s")

    @functools.partial(
        pl.kernel,
        out_type=jax.ShapeDtypeStruct((NC, n), jnp.float32),
        mesh=mesh,
        scratch_types=[
            pltpu.VMEM((nchunks, K), jnp.int32),
            pltpu.VMEM((nchunks, K), jnp.float32),
            pltpu.VMEM_SHARED((n,), jnp.float32),
        ],
        compiler_params=pltpu.CompilerParams(use_tc_tiling_on_sc=False),
    )
    def k(col_hbm, ew_hbm, z_hbm, out_hbm, cidx, ewv, acc):
        c = lax.axis_index("c")
        s = lax.axis_index("s")
        wid = s * NC + c

        @pl.when(s == 0)
        def _():
            pltpu.sync_copy(z_hbm, acc)

        plsc.subcore_barrier()
        pltpu.sync_copy(col_hbm.at[wid], cidx)
        pltpu.sync_copy(ew_hbm.at[wid], ewv)

        def chunk(i, carry):
            pltpu.sync_copy(ewv.at[i], acc.at[cidx.at[i]], add=True)
            return carry

        lax.fori_loop(0, nchunks, chunk, 0)
        plsc.subcore_barrier()

        @pl.when(s == 0)
        def _():
            pltpu.sync_copy(acc, out_hbm.at[c])

    return k(col3, ew3, zeros1)


def _sc_msg(row3, col3, ew3, y2, zeros2):
    _, n, dh = y2.shape
    np_ = zeros2.shape[0]
    nchunks = row3.shape[1]
    rows_per_tile = np_ // NS
    mesh = plsc.VectorSubcoreMesh(core_axis_name="c", subcore_axis_name="s")

    @functools.partial(
        pl.kernel,
        out_type=jax.ShapeDtypeStruct((NC, np_, dh), jnp.float32),
        mesh=mesh,
        scratch_types=[
            pltpu.VMEM((nchunks, K), jnp.int32),
            pltpu.VMEM((nchunks, K), jnp.int32),
            pltpu.VMEM((nchunks, K), jnp.float32),
            pltpu.VMEM((K, dh), jnp.float32),
            pltpu.VMEM((K, dh), jnp.float32),
            pltpu.VMEM((K, dh), jnp.float32),
            pltpu.VMEM((K, dh), jnp.float32),
            pltpu.VMEM_SHARED((np_, dh), jnp.float32),
            pltpu.SemaphoreType.DMA,
            pltpu.SemaphoreType.DMA,
        ],
        compiler_params=pltpu.CompilerParams(use_tc_tiling_on_sc=False),
    )
    def k(row_hbm, col_hbm, ew_hbm, y_hbm, z2_hbm, out_hbm,
          ridx, cidx, ewv, g0, g1, s0, s1, acc, gsem, ssem):
        c = lax.axis_index("c")
        s = lax.axis_index("s")
        rbase = s * rows_per_tile

        pltpu.sync_copy(z2_hbm.at[pl.ds(rbase, rows_per_tile)],
                        acc.at[pl.ds(rbase, rows_per_tile)])
        pltpu.sync_copy(row_hbm.at[s], ridx)
        pltpu.sync_copy(col_hbm.at[s], cidx)
        pltpu.sync_copy(ew_hbm.at[s], ewv)
        plsc.subcore_barrier()

        def issue_gather(i, gb):
            pltpu.async_copy(y_hbm.at[c].at[ridx.at[i]], gb, gsem)

        def wait_gather(i, gb):
            pltpu.make_async_copy(y_hbm.at[c].at[ridx.at[i]], gb, gsem).wait()

        def issue_scatter(i, sb):
            pltpu.async_copy(sb, acc.at[cidx.at[i]], ssem, add=True)

        def wait_scatter(i, sb):
            pltpu.make_async_copy(sb, acc.at[cidx.at[i]], ssem).wait()

        def scale(i, gb, sb):
            def sgroup(g, cc):
                base = g * LANES
                sv = ewv[i, pl.ds(base, LANES)]
                for l in range(LANES):
                    sc = sv[l]
                    r = base + l
                    for j in range(dh // LANES):
                        sl = (r, pl.ds(j * LANES, LANES))
                        sb[sl] = gb[sl] * sc
                return cc

            lax.fori_loop(0, K // LANES, sgroup, 0)

        issue_gather(0, g0)
        issue_gather(1, g1)

        def pair(gq, carry):
            for i, gb, sb in ((gq * 2, g0, s0), (gq * 2 + 1, g1, s1)):
                wait_gather(i, gb)

                @pl.when(gq > 0)
                def _():
                    wait_scatter(i, sb)

                scale(i, gb, sb)
                issue_gather(i + 2, gb)
                issue_scatter(i, sb)
            return carry

        lax.fori_loop(0, nchunks // 2 - 1, pair, 0)

        last = nchunks - 2
        for i, gb, sb in ((last, g0, s0), (last + 1, g1, s1)):
            wait_gather(i, gb)
            wait_scatter(i, sb)
            scale(i, gb, sb)
            issue_scatter(i, sb)
        wait_scatter(last, s0)
        wait_scatter(last + 1, s1)

        plsc.subcore_barrier()
        pltpu.sync_copy(acc.at[pl.ds(rbase, rows_per_tile)],
                        out_hbm.at[c, pl.ds(rbase, rows_per_tile)])

    return k(row3, col3, ew3, y2, zeros2)


def kernel(x, adj_t, edge_weight, W_lin, W_gcn):
    n, d = x.shape
    e = edge_weight.shape[0]
    nchunks = e // (NS * K)

    nchunks_deg = e // (NW * K)
    row_i = adj_t[0].astype(jnp.int32)
    col_i = adj_t[1].astype(jnp.int32)
    ew_f = edge_weight.astype(jnp.float32)
    col3d = col_i.reshape(NW, nchunks_deg, K)
    ew3d = ew_f.reshape(NW, nchunks_deg, K)
    row3 = row_i.reshape(NS, nchunks, K)
    col3 = col_i.reshape(NS, nchunks, K)
    ew3 = ew_f.reshape(NS, nchunks, K)
    zeros1 = jnp.zeros((n,), jnp.float32)
    npad = ((n + 8 * NS - 1) // (8 * NS)) * (8 * NS)
    zeros2 = jnp.zeros((npad, d // NC), jnp.float32)

    hl, xw = _tc_linear(x, W_lin, W_gcn)
    deg_p = _sc_deg(col3d, ew3d, zeros1)
    deg3 = deg_p.reshape(NC, n, 1)
    y2 = _tc_scale(deg3, xw)
    z = _sc_msg(row3, col3, ew3, y2, zeros2)
    return _tc_combine(hl, deg3, z)
